# baseline (speedup 1.0000x reference)
import jax
import jax.numpy as jnp
from jax.experimental import pallas as pl
from jax.experimental.pallas import tpu as pltpu

B = 1024
N = 512
D = 2048
INV_BETA = 20.0
CROSSCAM_EPOCH = 5
BG_KNN = 50
ROWS = 512
GRID = B // ROWS
NEG = -10000.0


def _bias16(v16):
    return (v16 ^ jnp.uint16(0x8000)).astype(jnp.int16)


def _sum16(a):
    n = a.shape[0]
    while n > 8:
        a = a[: n // 2] + a[n // 2:]
        n //= 2
    return jnp.sum(a.astype(jnp.int32), axis=0, keepdims=True)


def _body(epoch_ref, tgt_ref, feat_ref, em_ref, out_ref, s_ref, acc_ref):
    r = pl.program_id(0)
    j = pl.program_id(1)
    f = feat_ref[...]
    em = em_ref[...]
    part = jax.lax.dot_general(em, f, (((1,), (1,)), ((), ())),
                               preferred_element_type=jnp.float32)

    @pl.when(j == 0)
    def _lo():
        s_ref[:, : ROWS // 2] = part

    @pl.when(j == 1)
    def _hi():
        s_ref[:, ROWS // 2:] = part

    @pl.when(j == 1)
    def _run_tail():
        _tail(epoch_ref, tgt_ref, out_ref, s_ref, acc_ref, r)


def _tail(epoch_ref, tgt_ref, out_ref, s_ref, acc_ref, r):
    s = s_ref[...]
    tgt = tgt_ref[0]
    rows = jax.lax.broadcasted_iota(jnp.int32, (N, ROWS), 0)
    posmask = rows == tgt
    pos = jnp.sum(jnp.where(posmask, s, 0.0), axis=0, keepdims=True)

    m1 = jnp.max(s, axis=0, keepdims=True)
    exp_all = jnp.exp((s - m1) * INV_BETA)
    lse1 = m1 * INV_BETA + jnp.log(
        jnp.sum(exp_all, axis=0, keepdims=True))

    masked = jnp.where(posmask, NEG, s)
    u = jax.lax.bitcast_convert_type(masked, jnp.uint32)
    keys = u ^ jnp.where(u >> 31 != 0,
                         jnp.uint32(0xFFFFFFFF), jnp.uint32(0x80000000))
    k16 = _bias16((keys >> 16).astype(jnp.uint16))
    p16 = jnp.zeros((1, ROWS), jnp.uint32)
    c1 = jnp.full((1, ROWS), N, jnp.int32)
    one16 = jnp.int16(1)
    zero16 = jnp.int16(0)
    for bit in range(15, -1, -1):
        cand = p16 | jnp.uint32(1 << bit)
        cand16 = _bias16(cand.astype(jnp.uint16))
        cnt = _sum16(jnp.where(k16 >= cand16, one16, zero16))
        hit = cnt >= BG_KNN
        c1 = jnp.where(hit, cnt, c1)
        p16 = jnp.where(hit, cand, p16)
    t16 = _bias16(p16.astype(jnp.uint16))
    above = _sum16(jnp.where(k16 > t16, one16, zero16))
    lo16 = jnp.where(k16 == t16,
                     _bias16(keys.astype(jnp.uint16)), jnp.int16(-32768))
    plo = jnp.zeros((1, ROWS), jnp.uint32)
    need = BG_KNN - above
    inband = c1 - above
    for bit in range(15, -1, -1):
        cand = plo | jnp.uint32(1 << bit)
        cand16 = _bias16(cand.astype(jnp.uint16))
        cnt = _sum16(jnp.where(lo16 >= cand16, one16, zero16))
        hit = cnt >= need
        inband = jnp.where(hit, cnt, inband)
        plo = jnp.where(hit, cand, plo)
    prefix = (p16 << 16) | plo
    selmask = keys >= prefix
    nsel = (above + inband).astype(jnp.float32)
    tu = jnp.where(prefix >= jnp.uint32(0x80000000),
                   prefix ^ jnp.uint32(0x80000000), ~prefix)
    tval = jax.lax.bitcast_convert_type(tu, jnp.float32)

    z = m1 * INV_BETA
    sel = jnp.sum(jnp.where(selmask, exp_all, 0.0), axis=0, keepdims=True)
    sel = sel - (nsel - float(BG_KNN)) * jnp.exp(tval * INV_BETA - z)
    lse2 = z + jnp.log(jnp.exp(pos * INV_BETA - z) + sel)

    ce = jnp.sum(lse1 - pos * INV_BETA)
    assoc = jnp.sum(lse2 - pos * INV_BETA)

    @pl.when(r == 0)
    def _init():
        acc_ref[0, 0] = 0.0
        acc_ref[0, 1] = 0.0

    acc_ref[0, 0] += ce
    acc_ref[0, 1] += assoc

    @pl.when(r == GRID - 1)
    def _final():
        ce_t = acc_ref[0, 0] / float(B)
        as_t = acc_ref[0, 1]
        full = ce_t + 0.5 * as_t / float(B)
        out_ref[0, 0] = jnp.where(epoch_ref[0, 0] >= CROSSCAM_EPOCH,
                                  full, ce_t)


def kernel(features, targets, cams, epoch, all_pseudo_label, batch_ind,
           init_intra_id_feat):
    em = init_intra_id_feat[0]
    tgt3 = targets.reshape(GRID, 1, ROWS)
    ep = jnp.reshape(jnp.asarray(epoch, jnp.int32), (1, 1))
    loss = pl.pallas_call(
        _body,
        grid=(GRID, 2),
        in_specs=[
            pl.BlockSpec(memory_space=pltpu.SMEM),
            pl.BlockSpec((1, 1, ROWS), lambda r, j: (r, 0, 0)),
            pl.BlockSpec((ROWS // 2, D), lambda r, j: (2 * r + j, 0)),
            pl.BlockSpec((N, D), lambda r, j: (0, 0)),
        ],
        out_specs=pl.BlockSpec(memory_space=pltpu.SMEM),
        out_shape=jax.ShapeDtypeStruct((1, 1), jnp.float32),
        scratch_shapes=[pltpu.VMEM((N, ROWS), jnp.float32),
                        pltpu.SMEM((1, 2), jnp.float32)],
    )(ep, tgt3, features, em)
    return jnp.reshape(loss, (1,))


# speedup vs baseline: 1.1550x; 1.1550x over previous
import jax
import jax.numpy as jnp
from jax.experimental import pallas as pl
from jax.experimental.pallas import tpu as pltpu

B = 1024
N = 512
D = 2048
INV_BETA = 20.0
CROSSCAM_EPOCH = 5
BG_KNN = 50
ROWS = 512
GRID = B // ROWS
NEG = -10000.0


def _bias16(v16):
    return (v16 ^ jnp.uint16(0x8000)).astype(jnp.int16)


def _sum16(a):
    n = a.shape[0]
    while n > 8:
        a = a[: n // 2] + a[n // 2:]
        n //= 2
    return jnp.sum(a.astype(jnp.int32), axis=0, keepdims=True)


def _body(epoch_ref, tgt_ref, feat_ref, em_ref, out_ref, acc_ref):
    r = pl.program_id(0)
    f = feat_ref[...]
    em = em_ref[...]
    s = jax.lax.dot_general(em, f, (((1,), (1,)), ((), ())),
                            preferred_element_type=jnp.float32)
    tgt = tgt_ref[0]
    rows = jax.lax.broadcasted_iota(jnp.int32, (N, ROWS), 0)
    posmask = rows == tgt
    pos = jnp.sum(jnp.where(posmask, s, 0.0), axis=0, keepdims=True)

    m1 = jnp.max(s, axis=0, keepdims=True)
    exp_all = jnp.exp((s - m1) * INV_BETA)
    lse1 = m1 * INV_BETA + jnp.log(
        jnp.sum(exp_all, axis=0, keepdims=True))

    masked = jnp.where(posmask, NEG, s)
    u = jax.lax.bitcast_convert_type(masked, jnp.uint32)
    keys = u ^ jnp.where(u >> 31 != 0,
                         jnp.uint32(0xFFFFFFFF), jnp.uint32(0x80000000))
    k16 = _bias16((keys >> 16).astype(jnp.uint16))
    p16 = jnp.zeros((1, ROWS), jnp.uint32)
    c1 = jnp.full((1, ROWS), N, jnp.int32)
    one16 = jnp.int16(1)
    zero16 = jnp.int16(0)
    for bit in range(15, -1, -1):
        cand = p16 | jnp.uint32(1 << bit)
        cand16 = _bias16(cand.astype(jnp.uint16))
        cnt = _sum16(jnp.where(k16 >= cand16, one16, zero16))
        hit = cnt >= BG_KNN
        c1 = jnp.where(hit, cnt, c1)
        p16 = jnp.where(hit, cand, p16)
    t16 = _bias16(p16.astype(jnp.uint16))
    above = _sum16(jnp.where(k16 > t16, one16, zero16))
    lo16 = jnp.where(k16 == t16,
                     _bias16(keys.astype(jnp.uint16)), jnp.int16(-32768))
    plo = jnp.zeros((1, ROWS), jnp.uint32)
    need = BG_KNN - above
    inband = c1 - above
    for bit in range(15, -1, -1):
        cand = plo | jnp.uint32(1 << bit)
        cand16 = _bias16(cand.astype(jnp.uint16))
        cnt = _sum16(jnp.where(lo16 >= cand16, one16, zero16))
        hit = cnt >= need
        inband = jnp.where(hit, cnt, inband)
        plo = jnp.where(hit, cand, plo)
    prefix = (p16 << 16) | plo
    selmask = keys >= prefix
    nsel = (above + inband).astype(jnp.float32)
    tu = jnp.where(prefix >= jnp.uint32(0x80000000),
                   prefix ^ jnp.uint32(0x80000000), ~prefix)
    tval = jax.lax.bitcast_convert_type(tu, jnp.float32)

    z = m1 * INV_BETA
    sel = jnp.sum(jnp.where(selmask, exp_all, 0.0), axis=0, keepdims=True)
    sel = sel - (nsel - float(BG_KNN)) * jnp.exp(tval * INV_BETA - z)
    lse2 = z + jnp.log(jnp.exp(pos * INV_BETA - z) + sel)

    ce = jnp.sum(lse1 - pos * INV_BETA)
    assoc = jnp.sum(lse2 - pos * INV_BETA)

    @pl.when(r == 0)
    def _init():
        acc_ref[0, 0] = 0.0
        acc_ref[0, 1] = 0.0

    acc_ref[0, 0] += ce
    acc_ref[0, 1] += assoc

    @pl.when(r == GRID - 1)
    def _final():
        ce_t = acc_ref[0, 0] / float(B)
        as_t = acc_ref[0, 1]
        full = ce_t + 0.5 * as_t / float(B)
        out_ref[0, 0] = jnp.where(epoch_ref[0, 0] >= CROSSCAM_EPOCH,
                                  full, ce_t)


def kernel(features, targets, cams, epoch, all_pseudo_label, batch_ind,
           init_intra_id_feat):
    em = init_intra_id_feat[0]
    tgt3 = targets.reshape(GRID, 1, ROWS)
    ep = jnp.reshape(jnp.asarray(epoch, jnp.int32), (1, 1))
    loss = pl.pallas_call(
        _body,
        grid=(GRID,),
        in_specs=[
            pl.BlockSpec(memory_space=pltpu.SMEM),
            pl.BlockSpec((1, 1, ROWS), lambda i: (i, 0, 0)),
            pl.BlockSpec((ROWS, D), lambda i: (i, 0)),
            pl.BlockSpec((N, D), lambda i: (0, 0)),
        ],
        out_specs=pl.BlockSpec(memory_space=pltpu.SMEM),
        out_shape=jax.ShapeDtypeStruct((1, 1), jnp.float32),
        scratch_shapes=[pltpu.SMEM((1, 2), jnp.float32)],
    )(ep, tgt3, features, em)
    return jnp.reshape(loss, (1,))
